# Initial kernel scaffold; baseline (speedup 1.0000x reference)
import functools

import jax
import jax.numpy as jnp
from jax import lax
from jax.experimental import pallas as pl
from jax.experimental.pallas import tpu as pltpu
from jax.experimental.pallas import tpu_sc as plsc

NC = 2
NS = 16
NW = NC * NS

CHUNK = 100
STEPS = 64
PER_W = CHUNK * STEPS


def _body(table_hbm, idx_hbm, out_hbm, idx_v, rows0, rows1, g0, g1, w0, w1):
    wid = lax.axis_index("s") * NC + lax.axis_index("c")
    base = wid * PER_W
    rows = (rows0, rows1)
    gsem = (g0, g1)
    wsem = (w0, w1)

    pltpu.sync_copy(idx_hbm.at[wid], idx_v)

    pltpu.async_copy(table_hbm.at[idx_v.at[0]], rows0, g0)
    pltpu.async_copy(table_hbm.at[idx_v.at[1]], rows1, g1)

    def step(g, carry):
        for b in range(2):
            s = g * 2 + b
            pltpu.make_async_copy(table_hbm.at[idx_v.at[s]], rows[b], gsem[b]).wait()
            dst = out_hbm.at[pl.ds(base + s * CHUNK, CHUNK)]
            pltpu.async_copy(rows[b], dst, wsem[b])

            @pl.when(s + 2 < STEPS)
            def _():
                pltpu.make_async_copy(rows[b], dst, wsem[b]).wait()
                pltpu.async_copy(table_hbm.at[idx_v.at[s + 2]], rows[b], gsem[b])

        return carry

    lax.fori_loop(0, STEPS // 2, step, 0)

    last0 = out_hbm.at[pl.ds(base + (STEPS - 2) * CHUNK, CHUNK)]
    last1 = out_hbm.at[pl.ds(base + (STEPS - 1) * CHUNK, CHUNK)]
    pltpu.make_async_copy(rows0, last0, w0).wait()
    pltpu.make_async_copy(rows1, last1, w1).wait()


def kernel(ids, kernel):
    table = kernel
    n_rows, d = table.shape
    b_total = ids.shape[0] * ids.shape[1]
    assert b_total == NW * PER_W

    idx = ids.reshape(NW, STEPS, CHUNK).astype(jnp.int32)

    mesh = plsc.VectorSubcoreMesh(
        core_axis_name="c", subcore_axis_name="s", num_cores=NC, num_subcores=NS
    )
    run = pl.kernel(
        _body,
        out_type=jax.ShapeDtypeStruct((b_total, d), table.dtype),
        mesh=mesh,
        scratch_types=[
            pltpu.VMEM((STEPS, CHUNK), jnp.int32),
            pltpu.VMEM((CHUNK, d), jnp.float32),
            pltpu.VMEM((CHUNK, d), jnp.float32),
            pltpu.SemaphoreType.DMA,
            pltpu.SemaphoreType.DMA,
            pltpu.SemaphoreType.DMA,
            pltpu.SemaphoreType.DMA,
        ],
    )
    out = run(table, idx)
    return out.reshape(ids.shape[0], ids.shape[1], d)


# speedup vs baseline: 1.5621x; 1.5621x over previous
import functools

import jax
import jax.numpy as jnp
from jax import lax
from jax.experimental import pallas as pl
from jax.experimental.pallas import tpu as pltpu
from jax.experimental.pallas import tpu_sc as plsc

NC = 2
NS = 16
NW = NC * NS

CHUNK = 80
STEPS = 80
PER_W = CHUNK * STEPS


def _body(table_hbm, idx_hbm, out_hbm, idx_v, rows0, rows1, g0, g1, w0, w1):
    wid = lax.axis_index("s") * NC + lax.axis_index("c")
    base = wid * PER_W
    rows = (rows0, rows1)
    gsem = (g0, g1)
    wsem = (w0, w1)

    pltpu.sync_copy(idx_hbm.at[wid], idx_v)

    pltpu.async_copy(table_hbm.at[idx_v.at[0]], rows0, g0)
    pltpu.async_copy(table_hbm.at[idx_v.at[1]], rows1, g1)

    def step(g, carry):
        for b in range(2):
            s = g * 2 + b
            pltpu.make_async_copy(table_hbm.at[idx_v.at[s]], rows[b], gsem[b]).wait()
            dst = out_hbm.at[pl.ds(base + s * CHUNK, CHUNK)]
            pltpu.async_copy(rows[b], dst, wsem[b])

            @pl.when(s + 2 < STEPS)
            def _():
                pltpu.make_async_copy(rows[b], dst, wsem[b]).wait()
                pltpu.async_copy(table_hbm.at[idx_v.at[s + 2]], rows[b], gsem[b])

        return carry

    lax.fori_loop(0, STEPS // 2, step, 0)

    last0 = out_hbm.at[pl.ds(base + (STEPS - 2) * CHUNK, CHUNK)]
    last1 = out_hbm.at[pl.ds(base + (STEPS - 1) * CHUNK, CHUNK)]
    pltpu.make_async_copy(rows0, last0, w0).wait()
    pltpu.make_async_copy(rows1, last1, w1).wait()


def kernel(ids, kernel):
    table = kernel
    n_rows, d = table.shape
    b_total = ids.shape[0] * ids.shape[1]
    assert b_total == NW * PER_W

    idx = ids.reshape(NW, STEPS, CHUNK).astype(jnp.int32)

    mesh = plsc.VectorSubcoreMesh(
        core_axis_name="c", subcore_axis_name="s", num_cores=NC, num_subcores=NS
    )
    run = pl.kernel(
        _body,
        out_type=jax.ShapeDtypeStruct((b_total, d), table.dtype),
        mesh=mesh,
        scratch_types=[
            pltpu.VMEM((STEPS, CHUNK), jnp.int32),
            pltpu.VMEM((CHUNK, d), jnp.float32),
            pltpu.VMEM((CHUNK, d), jnp.float32),
            pltpu.SemaphoreType.DMA,
            pltpu.SemaphoreType.DMA,
            pltpu.SemaphoreType.DMA,
            pltpu.SemaphoreType.DMA,
        ],
    )
    out = run(table, idx)
    return out.reshape(ids.shape[0], ids.shape[1], d)


# speedup vs baseline: 1.5629x; 1.0005x over previous
import functools

import jax
import jax.numpy as jnp
from jax import lax
from jax.experimental import pallas as pl
from jax.experimental.pallas import tpu as pltpu
from jax.experimental.pallas import tpu_sc as plsc

NC = 2
NS = 16
NW = NC * NS

CHUNK = 80
STEPS = 80
PER_W = CHUNK * STEPS


def _body(table_hbm, idx_hbm, out_hbm, idx_v, rows0, rows1, g0, g1, w0, w1):
    sid = lax.axis_index("s")
    wid = sid * NC + lax.axis_index("c")
    base = wid * PER_W
    rows = (rows0, rows1)
    gsem = (g0, g1)
    wsem = (w0, w1)

    pltpu.sync_copy(idx_hbm.at[wid], idx_v)

    pltpu.async_copy(table_hbm.at[idx_v.at[0]], rows0, g0)
    pltpu.async_copy(table_hbm.at[idx_v.at[1]], rows1, g1)

    def step(g, carry):
        for b in range(2):
            s = g * 2 + b
            pltpu.make_async_copy(table_hbm.at[idx_v.at[s]], rows[b], gsem[b]).wait()
            dst = out_hbm.at[pl.ds(base + s * CHUNK, CHUNK)]
            pltpu.async_copy(rows[b], dst, wsem[b])

            @pl.when(s + 2 < STEPS)
            def _():
                pltpu.make_async_copy(rows[b], dst, wsem[b]).wait()
                pltpu.async_copy(table_hbm.at[idx_v.at[s + 2]], rows[b], gsem[b])

        return carry

    lax.fori_loop(0, STEPS // 2, step, 0)

    last0 = out_hbm.at[pl.ds(base + (STEPS - 2) * CHUNK, CHUNK)]
    last1 = out_hbm.at[pl.ds(base + (STEPS - 1) * CHUNK, CHUNK)]
    pltpu.make_async_copy(rows0, last0, w0).wait()
    pltpu.make_async_copy(rows1, last1, w1).wait()


def kernel(ids, kernel):
    table = kernel
    n_rows, d = table.shape
    b_total = ids.shape[0] * ids.shape[1]
    assert b_total == NW * PER_W

    idx = ids.reshape(NW, STEPS, CHUNK).astype(jnp.int32)

    mesh = plsc.VectorSubcoreMesh(
        core_axis_name="c", subcore_axis_name="s", num_cores=NC, num_subcores=NS
    )
    run = pl.kernel(
        _body,
        out_type=jax.ShapeDtypeStruct((b_total, d), table.dtype),
        mesh=mesh,
        scratch_types=[
            pltpu.VMEM((STEPS, CHUNK), jnp.int32),
            pltpu.VMEM((CHUNK, d), jnp.float32),
            pltpu.VMEM((CHUNK, d), jnp.float32),
            pltpu.SemaphoreType.DMA,
            pltpu.SemaphoreType.DMA,
            pltpu.SemaphoreType.DMA,
            pltpu.SemaphoreType.DMA,
        ],
    )
    out = run(table, idx)
    return out.reshape(ids.shape[0], ids.shape[1], d)


# speedup vs baseline: 2.1951x; 1.4045x over previous
import jax
import jax.numpy as jnp
from jax import lax
from jax.experimental import pallas as pl
from jax.experimental.pallas import tpu as pltpu
from jax.experimental.pallas import tpu_sc as plsc

NC = 2
NS = 16
NW = NC * NS

NB = 2
ROWS_W = 128
STEPS = ROWS_W // NB


def _body(table_hbm, idx_hbm, out_hbm, idx_v, st0, st1, g0, g1, w0, w1):
    wid = lax.axis_index("s") * NC + lax.axis_index("c")
    row0 = wid * ROWS_W
    stage = (st0, st1)
    gsem = (g0, g1)
    wsem = (w0, w1)

    pltpu.sync_copy(idx_hbm.at[pl.ds(row0, ROWS_W)], idx_v)

    def gathers(s, b):
        for j in range(NB):
            pltpu.async_copy(
                table_hbm.at[idx_v.at[s * NB + j]], stage[b].at[j], gsem[b]
            )

    def wait_gathers(s, b):
        for j in range(NB):
            pltpu.make_async_copy(
                table_hbm.at[idx_v.at[s * NB + j]], stage[b].at[j], gsem[b]
            ).wait()

    gathers(0, 0)
    gathers(1, 1)

    def step(g, carry):
        for b in range(2):
            s = g * 2 + b
            wait_gathers(s, b)
            dst = out_hbm.at[pl.ds(row0 + s * NB, NB)]
            pltpu.async_copy(stage[b], dst, wsem[b])

            @pl.when(s + 2 < STEPS)
            def _():
                pltpu.make_async_copy(stage[b], dst, wsem[b]).wait()
                gathers(s + 2, b)

        return carry

    lax.fori_loop(0, STEPS // 2, step, 0)

    last0 = out_hbm.at[pl.ds(row0 + (STEPS - 2) * NB, NB)]
    last1 = out_hbm.at[pl.ds(row0 + (STEPS - 1) * NB, NB)]
    pltpu.make_async_copy(st0, last0, w0).wait()
    pltpu.make_async_copy(st1, last1, w1).wait()


def kernel(ids, kernel):
    table = kernel
    n_rows, d = table.shape
    nb_rows, seq = ids.shape
    assert nb_rows == NW * ROWS_W

    idx = ids.astype(jnp.int32)

    mesh = plsc.VectorSubcoreMesh(
        core_axis_name="c", subcore_axis_name="s", num_cores=NC, num_subcores=NS
    )
    run = pl.kernel(
        _body,
        out_type=jax.ShapeDtypeStruct((nb_rows, seq, d), table.dtype),
        mesh=mesh,
        scratch_types=[
            pltpu.VMEM((ROWS_W, seq), jnp.int32),
            pltpu.VMEM((NB, seq, d), jnp.float32),
            pltpu.VMEM((NB, seq, d), jnp.float32),
            pltpu.SemaphoreType.DMA,
            pltpu.SemaphoreType.DMA,
            pltpu.SemaphoreType.DMA,
            pltpu.SemaphoreType.DMA,
        ],
    )
    return run(table, idx)


# speedup vs baseline: 3.7445x; 1.7059x over previous
import jax
import jax.numpy as jnp
from jax import lax
from jax.experimental import pallas as pl
from jax.experimental.pallas import tpu as pltpu
from jax.experimental.pallas import tpu_sc as plsc

NC = 2
NS = 16
NW = NC * NS

ROWS_W = 128
HALF = 64
UNITS = 50 * (ROWS_W // HALF)
NBUF = 3
UNITS_EFF = 102


def _body(table_hbm, idx_hbm, out_hbm, idx_v, st0, st1, st2, g0, g1, g2, w0, w1, w2):
    wid = lax.axis_index("s") * NC + lax.axis_index("c")
    col0 = wid * ROWS_W
    stage = (st0, st1, st2)
    gsem = (g0, g1, g2)
    wsem = (w0, w1, w2)

    pltpu.sync_copy(idx_hbm.at[wid], idx_v)

    def dst_of(u):
        ur = lax.rem(u, UNITS)
        j = ur // 2
        h = lax.rem(ur, 2)
        return out_hbm.at[j, pl.ds(col0 + h * HALF, HALF)]

    for b in range(NBUF):
        pltpu.async_copy(table_hbm.at[idx_v.at[b]], stage[b], gsem[b])

    def step(g, carry):
        for b in range(NBUF):
            u = g * NBUF + b
            pltpu.make_async_copy(table_hbm.at[idx_v.at[u]], stage[b], gsem[b]).wait()
            dst = dst_of(u)
            pltpu.async_copy(stage[b], dst, wsem[b])

            @pl.when(u + NBUF < UNITS_EFF)
            def _():
                pltpu.make_async_copy(stage[b], dst, wsem[b]).wait()
                pltpu.async_copy(table_hbm.at[idx_v.at[u + NBUF]], stage[b], gsem[b])

        return carry

    lax.fori_loop(0, UNITS_EFF // NBUF, step, 0)

    for b in range(NBUF):
        u = UNITS_EFF - NBUF + b
        pltpu.make_async_copy(stage[b], dst_of(u), wsem[b]).wait()


def kernel(ids, kernel):
    table = kernel
    n_rows, d = table.shape
    nb_rows, seq = ids.shape
    assert nb_rows == NW * ROWS_W

    idx = (
        ids.astype(jnp.int32)
        .T.reshape(seq, NW, ROWS_W // HALF, HALF)
        .transpose(1, 0, 2, 3)
        .reshape(NW, UNITS, HALF)
    )
    idx = jnp.concatenate([idx, idx[:, : UNITS_EFF - UNITS, :]], axis=1)

    mesh = plsc.VectorSubcoreMesh(
        core_axis_name="c", subcore_axis_name="s", num_cores=NC, num_subcores=NS
    )
    run = pl.kernel(
        _body,
        out_type=jax.ShapeDtypeStruct((seq, nb_rows, d), table.dtype),
        mesh=mesh,
        scratch_types=[
            pltpu.VMEM((UNITS_EFF, HALF), jnp.int32),
            pltpu.VMEM((HALF, d), jnp.float32),
            pltpu.VMEM((HALF, d), jnp.float32),
            pltpu.VMEM((HALF, d), jnp.float32),
            pltpu.SemaphoreType.DMA,
            pltpu.SemaphoreType.DMA,
            pltpu.SemaphoreType.DMA,
            pltpu.SemaphoreType.DMA,
            pltpu.SemaphoreType.DMA,
            pltpu.SemaphoreType.DMA,
        ],
    )
    out3 = run(table, idx)
    return out3.transpose(1, 0, 2)
